# baseline (speedup 1.0000x reference)
import functools

import jax
import jax.numpy as jnp
from jax import lax
from jax.experimental import pallas as pl
from jax.experimental.pallas import tpu as pltpu
from jax.experimental.pallas import tpu_sc as plsc

NC = 2
NS = 16
LANES = 16
NW = NC * NS

G = 128
K = 2
C = G * K


@functools.partial(jax.jit, static_argnums=(3, 4))
def _embed(token_table, seg_plus, comb, N, E):
    npw = N // NW
    n_chunks = npw // C
    assert n_chunks % 2 == 0
    EV = E // LANES

    mesh = plsc.VectorSubcoreMesh(
        core_axis_name="c", subcore_axis_name="s", num_cores=NC, num_subcores=NS
    )

    @functools.partial(
        pl.kernel,
        out_type=jax.ShapeDtypeStruct((N, E), jnp.float32),
        mesh=mesh,
        scratch_types=[
            pltpu.VMEM((npw,), jnp.int32),
            pltpu.VMEM((K, G), jnp.int32),
            pltpu.VMEM((K, G), jnp.int32),
            pltpu.VMEM((K, G), jnp.int32),
            pltpu.VMEM((K, G), jnp.int32),
            pltpu.VMEM((C, E), jnp.float32),
            pltpu.VMEM((C, E), jnp.float32),
            pltpu.VMEM((C, E), jnp.float32),
            pltpu.VMEM((C, E), jnp.float32),
            pltpu.SemaphoreType.DMA,
            pltpu.SemaphoreType.DMA,
            pltpu.SemaphoreType.DMA,
            pltpu.SemaphoreType.DMA,
        ],
        compiler_params=pltpu.CompilerParams(use_tc_tiling_on_sc=False),
    )
    def k(tok_hbm, seg_hbm, comb_hbm, out_hbm,
          comb_v, idx0, idx1, lidx0, lidx1, rows0, rows1, srows0, srows1,
          sg0, sg1, sw0, sw1):
        idx_b = (idx0, idx1)
        lidx_b = (lidx0, lidx1)
        rows_b = (rows0, rows1)
        srows_b = (srows0, srows1)
        sg_b = (sg0, sg1)
        sw_b = (sw0, sw1)
        wid = lax.axis_index("s") * NC + lax.axis_index("c")
        base0 = pl.multiple_of(wid * npw, C)
        pltpu.sync_copy(comb_hbm.at[pl.ds(base0, npw)], comb_v)

        def decode(i, buf):
            off = i * C
            for t in range(C // LANES):
                v = comb_v[pl.ds(off + t * LANES, LANES)]
                idx_b[buf][(t * LANES) // G, pl.ds((t * LANES) % G, LANES)] = (
                    lax.shift_right_logical(v, 2)
                )
                lidx_b[buf][(t * LANES) // G, pl.ds((t * LANES) % G, LANES)] = (
                    lax.bitwise_and(v, 3)
                )

        def fire_gathers(buf, i):
            base = pl.multiple_of(base0 + i * C, C)
            for j in range(K):
                pltpu.async_copy(
                    tok_hbm.at[idx_b[buf].at[j]],
                    rows_b[buf].at[pl.ds(j * G, G)],
                    sg_b[buf],
                )
                pltpu.async_copy(
                    seg_hbm.at[lidx_b[buf].at[j]],
                    srows_b[buf].at[pl.ds(j * G, G)],
                    sg_b[buf],
                )

        def drain(sem, ref):
            pltpu.make_async_copy(tok_hbm.at[pl.ds(0, C)], ref, sem).wait()

        def seg_add(i, buf):
            rows = rows_b[buf]
            srows = srows_b[buf]

            def row_body(r, c2):
                for jj in range(EV):
                    sl = pl.ds(jj * LANES, LANES)
                    rows[r, sl] = rows[r, sl] + srows[r, sl]
                return c2

            lax.fori_loop(0, C, row_body, 0, unroll=4)

        def fire_write(i, buf):
            base = pl.multiple_of(base0 + i * C, C)
            pltpu.async_copy(rows_b[buf], out_hbm.at[pl.ds(base, C)], sw_b[buf])

        decode(0, 0)
        fire_gathers(0, 0)

        def pair_body(t, carry):
            i = t * 2
            decode(i + 1, 1)

            @pl.when(t > 0)
            def _():
                drain(sw_b[1], rows_b[1])
            fire_gathers(1, i + 1)
            drain(sg_b[0], rows_b[0])
            drain(sg_b[0], srows_b[0])
            seg_add(i, 0)
            fire_write(i, 0)

            @pl.when(t < n_chunks // 2 - 1)
            def _():
                decode(i + 2, 0)
                drain(sw_b[0], rows_b[0])
                fire_gathers(0, i + 2)
            drain(sg_b[1], rows_b[1])
            drain(sg_b[1], srows_b[1])
            seg_add(i + 1, 1)
            fire_write(i + 1, 1)
            return carry

        lax.fori_loop(0, n_chunks // 2, pair_body, 0)
        drain(sw_b[0], rows_b[0])
        drain(sw_b[1], rows_b[1])

    return k(token_table, seg_plus, comb)


def kernel(token_table, segment_table, pe, sequence, segment_label):
    B, L = sequence.shape
    V, E = token_table.shape
    N = B * L
    pos = pe[0, L]
    seg_plus = segment_table + pos[None, :]
    seg_plus = jnp.concatenate([seg_plus, jnp.zeros((1, E), seg_plus.dtype)], 0)
    comb = (sequence * 4 + segment_label).reshape(N)
    out = _embed(token_table, seg_plus, comb, N, E)
    return out.reshape(B, L, E)


# speedup vs baseline: 8.8651x; 8.8651x over previous
import functools

import jax
import jax.numpy as jnp
from jax import lax
from jax.experimental import pallas as pl
from jax.experimental.pallas import tpu as pltpu
from jax.experimental.pallas import tpu_sc as plsc

NC = 2
NS = 16
LANES = 16
NW = NC * NS

G = 128
K = 2
C = G * K
EP = 128


@functools.partial(jax.jit, static_argnums=(3, 4))
def _embed(tok_pad, seg_plus, comb, N, E):
    npw = N // NW
    n_chunks = npw // C
    assert n_chunks % 2 == 0
    EV = E // LANES

    mesh = plsc.VectorSubcoreMesh(
        core_axis_name="c", subcore_axis_name="s", num_cores=NC, num_subcores=NS
    )

    @functools.partial(
        pl.kernel,
        out_type=jax.ShapeDtypeStruct((N, EP), jnp.float32),
        mesh=mesh,
        scratch_types=[
            pltpu.VMEM((npw,), jnp.int32),
            pltpu.VMEM((K, G), jnp.int32),
            pltpu.VMEM((K, G), jnp.int32),
            pltpu.VMEM((C, EP), jnp.float32),
            pltpu.VMEM((C, EP), jnp.float32),
            pltpu.VMEM((4 * E,), jnp.float32),
            pltpu.SemaphoreType.DMA,
            pltpu.SemaphoreType.DMA,
            pltpu.SemaphoreType.DMA,
            pltpu.SemaphoreType.DMA,
        ],
        compiler_params=pltpu.CompilerParams(use_tc_tiling_on_sc=False),
    )
    def k(tok_hbm, seg_hbm, comb_hbm, out_hbm,
          comb_v, idx0, idx1, rows0, rows1, seg_v,
          sg0, sg1, sw0, sw1):
        idx_b = (idx0, idx1)
        rows_b = (rows0, rows1)
        sg_b = (sg0, sg1)
        sw_b = (sw0, sw1)
        wid = lax.axis_index("s") * NC + lax.axis_index("c")
        base0 = pl.multiple_of(wid * npw, C)
        pltpu.sync_copy(seg_hbm, seg_v)
        pltpu.sync_copy(comb_hbm.at[pl.ds(base0, npw)], comb_v)

        def decode(i, buf):
            off = i * C
            for t in range(C // LANES):
                v = comb_v[pl.ds(off + t * LANES, LANES)]
                idx_b[buf][(t * LANES) // G, pl.ds((t * LANES) % G, LANES)] = (
                    lax.shift_right_logical(v, 2)
                )

        def fire_gathers(buf, i):
            for j in range(K):
                pltpu.async_copy(
                    tok_hbm.at[idx_b[buf].at[j]],
                    rows_b[buf].at[pl.ds(j * G, G)],
                    sg_b[buf],
                )

        def drain(sem, ref):
            pltpu.make_async_copy(tok_hbm.at[pl.ds(0, C)], ref, sem).wait()

        def seg_add(i, buf):
            rows = rows_b[buf]
            off = i * C

            def row_body(t, c2):
                labv = comb_v[pl.ds(off + t * LANES, LANES)]
                for kk in range(LANES):
                    soff = lax.bitwise_and(labv[kk], 3) * E
                    r = t * LANES + kk
                    for jj in range(EV):
                        sl = pl.ds(jj * LANES, LANES)
                        sv = seg_v[pl.ds(soff + jj * LANES, LANES)]
                        rows[r, sl] = rows[r, sl] + sv
                return c2

            lax.fori_loop(0, C // LANES, row_body, 0)

        def fire_write(i, buf):
            base = pl.multiple_of(base0 + i * C, C)
            pltpu.async_copy(rows_b[buf], out_hbm.at[pl.ds(base, C)], sw_b[buf])

        decode(0, 0)
        fire_gathers(0, 0)

        def pair_body(t, carry):
            i = t * 2
            decode(i + 1, 1)

            @pl.when(t > 0)
            def _():
                drain(sw_b[1], rows_b[1])
            fire_gathers(1, i + 1)
            drain(sg_b[0], rows_b[0])
            seg_add(i, 0)
            fire_write(i, 0)

            @pl.when(t < n_chunks // 2 - 1)
            def _():
                decode(i + 2, 0)
                drain(sw_b[0], rows_b[0])
                fire_gathers(0, i + 2)
            drain(sg_b[1], rows_b[1])
            seg_add(i + 1, 1)
            fire_write(i + 1, 1)
            return carry

        lax.fori_loop(0, n_chunks // 2, pair_body, 0)
        drain(sw_b[0], rows_b[0])
        drain(sw_b[1], rows_b[1])

    return k(tok_pad, seg_plus, comb)


def kernel(token_table, segment_table, pe, sequence, segment_label):
    B, L = sequence.shape
    V, E = token_table.shape
    N = B * L
    pos = pe[0, L]
    seg_plus = segment_table + pos[None, :]
    seg_plus = jnp.concatenate([seg_plus, jnp.zeros((1, E), seg_plus.dtype)], 0).reshape(-1)
    comb = (sequence * 4 + segment_label).reshape(N)
    tok_pad = jnp.pad(token_table, ((0, 0), (0, EP - E)))
    out = _embed(tok_pad, seg_plus, comb, N, E)
    return out.reshape(B, L, EP)[:, :, :E]


# speedup vs baseline: 8.8815x; 1.0018x over previous
import functools

import jax
import jax.numpy as jnp
from jax import lax
from jax.experimental import pallas as pl
from jax.experimental.pallas import tpu as pltpu
from jax.experimental.pallas import tpu_sc as plsc

NC = 2
NS = 16
LANES = 16
NW = NC * NS

G = 128
K = 2
C = G * K
EP = 128


@functools.partial(jax.jit, static_argnums=(3, 4))
def _embed(tok_pad, seg_plus, comb, N, E):
    npw = N // NW
    n_chunks = npw // C
    assert n_chunks % 2 == 0
    EV = E // LANES

    mesh = plsc.VectorSubcoreMesh(
        core_axis_name="c", subcore_axis_name="s", num_cores=NC, num_subcores=NS
    )

    @functools.partial(
        pl.kernel,
        out_type=jax.ShapeDtypeStruct((N, EP), jnp.float32),
        mesh=mesh,
        scratch_types=[
            pltpu.VMEM((npw,), jnp.int32),
            pltpu.VMEM((K, G), jnp.int32),
            pltpu.VMEM((K, G), jnp.int32),
            pltpu.VMEM((C, EP), jnp.float32),
            pltpu.VMEM((C, EP), jnp.float32),
            pltpu.VMEM((4 * E,), jnp.float32),
            pltpu.SemaphoreType.DMA,
            pltpu.SemaphoreType.DMA,
            pltpu.SemaphoreType.DMA,
            pltpu.SemaphoreType.DMA,
        ],
        compiler_params=pltpu.CompilerParams(use_tc_tiling_on_sc=False),
    )
    def k(tok_hbm, seg_hbm, comb_hbm, out_hbm,
          comb_v, idx0, idx1, rows0, rows1, seg_v,
          sg0, sg1, sw0, sw1):
        idx_b = (idx0, idx1)
        rows_b = (rows0, rows1)
        sg_b = (sg0, sg1)
        sw_b = (sw0, sw1)
        wid = lax.axis_index("s") * NC + lax.axis_index("c")
        base0 = pl.multiple_of(wid * npw, C)
        pltpu.sync_copy(seg_hbm, seg_v)
        pltpu.sync_copy(comb_hbm.at[pl.ds(base0, npw)], comb_v)

        def decode(i, buf):
            off = i * C
            for t in range(C // LANES):
                v = comb_v[pl.ds(off + t * LANES, LANES)]
                idx_b[buf][(t * LANES) // G, pl.ds((t * LANES) % G, LANES)] = (
                    lax.shift_right_logical(v, 2)
                )

        def fire_gathers(buf, i):
            for j in range(K):
                pltpu.async_copy(
                    tok_hbm.at[idx_b[buf].at[j]],
                    rows_b[buf].at[pl.ds(j * G, G)],
                    sg_b[buf],
                )

        def drain(sem, ref):
            pltpu.make_async_copy(tok_hbm.at[pl.ds(0, C)], ref, sem).wait()

        def seg_add(i, buf):
            rows = rows_b[buf]
            off = i * C
            d1 = [seg_v[pl.ds(1 * E + jj * LANES, LANES)] for jj in range(EV)]
            d2 = [seg_v[pl.ds(2 * E + jj * LANES, LANES)] for jj in range(EV)]

            def row_body(t, c2):
                v = comb_v[pl.ds(off + t * LANES, LANES)]
                a1v = lax.convert_element_type(lax.bitwise_and(v, 1), jnp.float32)
                a2v = lax.convert_element_type(
                    lax.bitwise_and(lax.shift_right_logical(v, 1), 1), jnp.float32)
                for kk in range(LANES):
                    b1 = lax.broadcast(a1v[kk], (LANES,))
                    b2 = lax.broadcast(a2v[kk], (LANES,))
                    r = t * LANES + kk
                    for jj in range(EV):
                        sl = pl.ds(jj * LANES, LANES)
                        rows[r, sl] = rows[r, sl] + b1 * d1[jj] + b2 * d2[jj]
                return c2

            lax.fori_loop(0, C // LANES, row_body, 0)

        def fire_write(i, buf):
            base = pl.multiple_of(base0 + i * C, C)
            pltpu.async_copy(rows_b[buf], out_hbm.at[pl.ds(base, C)], sw_b[buf])

        decode(0, 0)
        fire_gathers(0, 0)

        def pair_body(t, carry):
            i = t * 2
            decode(i + 1, 1)

            @pl.when(t > 0)
            def _():
                drain(sw_b[1], rows_b[1])
            fire_gathers(1, i + 1)
            drain(sg_b[0], rows_b[0])
            seg_add(i, 0)
            fire_write(i, 0)

            @pl.when(t < n_chunks // 2 - 1)
            def _():
                decode(i + 2, 0)
                drain(sw_b[0], rows_b[0])
                fire_gathers(0, i + 2)
            drain(sg_b[1], rows_b[1])
            seg_add(i + 1, 1)
            fire_write(i + 1, 1)
            return carry

        lax.fori_loop(0, n_chunks // 2, pair_body, 0)
        drain(sw_b[0], rows_b[0])
        drain(sw_b[1], rows_b[1])

    return k(tok_pad, seg_plus, comb)


def kernel(token_table, segment_table, pe, sequence, segment_label):
    B, L = sequence.shape
    V, E = token_table.shape
    N = B * L
    pos = pe[0, L]
    seg_flat = jnp.concatenate(
        [segment_table, jnp.zeros((1, E), segment_table.dtype)], 0).reshape(-1)
    comb = (sequence * 4 + segment_label).reshape(N)
    tok_pad = jnp.pad(token_table + pos[None, :], ((0, 0), (0, EP - E)))
    out = _embed(tok_pad, seg_flat, comb, N, E)
    return out.reshape(B, L, EP)[:, :, :E]
